# Initial kernel scaffold; baseline (speedup 1.0000x reference)
import jax
import jax.numpy as jnp
from jax.experimental import pallas as pl


def kernel(x, edge_index, W1, b1, W2, b2, W3, b3):
    raise NotImplementedError("write your pallas kernel here")


# speedup vs baseline: 7.3694x; 7.3694x over previous
import functools

import jax
import jax.numpy as jnp
from jax import lax
from jax.experimental import pallas as pl
from jax.experimental.pallas import tpu as pltpu
from jax.experimental.pallas import tpu_sc as plsc

N = 10000
D = 128
E = 320000
NC = 2
NS = 16
NW = NC * NS
EPW = E // NW
K = 128
EPW_PAD = 10240
NCHUNK = EPW_PAD // K
PAD_N = 10240
ROWS_PER_TILE = PAD_N // NS
ZROWS = 64

BLK = 1000
GRID = N // BLK

_mesh = plsc.VectorSubcoreMesh(core_axis_name="c", subcore_axis_name="s")
_sc_params = pltpu.CompilerParams(needs_layout_passes=False)


@functools.partial(
    pl.kernel,
    mesh=_mesh,
    out_type=jax.ShapeDtypeStruct((NW, N), jnp.float32),
    scratch_types=[
        pltpu.VMEM((EPW,), jnp.int32),
        pltpu.VMEM((N,), jnp.float32),
    ],
    compiler_params=_sc_params,
)
def _deg_kernel(dst_hbm, out_hbm, idx_v, deg_v):
    c = lax.axis_index("c")
    s = lax.axis_index("s")
    wid = s * NC + c
    pltpu.sync_copy(dst_hbm.at[pl.ds(wid * EPW, EPW)], idx_v)
    zeros16 = jnp.zeros((16,), jnp.float32)
    ones16 = jnp.ones((16,), jnp.float32)

    def zbody(i, carry):
        deg_v[pl.ds(i * 16, 16)] = zeros16
        return carry

    lax.fori_loop(0, N // 16, zbody, 0)

    def body(i, carry):
        idx = idx_v[pl.ds(i * 16, 16)]
        plsc.addupdate_scatter(deg_v, [idx], ones16)
        return carry

    lax.fori_loop(0, EPW // 16, body, 0)
    pltpu.sync_copy(deg_v, out_hbm.at[wid])


@functools.partial(
    pl.kernel,
    mesh=_mesh,
    out_type=jax.ShapeDtypeStruct((NC, PAD_N, D), jnp.float32),
    scratch_types=[
        pltpu.VMEM((K,), jnp.int32),
        pltpu.VMEM((K,), jnp.int32),
        pltpu.VMEM((K, D), jnp.float32),
        pltpu.VMEM((ZROWS, D), jnp.float32),
        pltpu.VMEM_SHARED((PAD_N, D), jnp.float32),
        pltpu.SemaphoreType.DMA,
    ],
    compiler_params=_sc_params,
)
def _scatter_kernel(g_hbm, src_hbm, dst_hbm, out_hbm,
                    sidx, didx, rows, zbuf, acc, sem):
    c = lax.axis_index("c")
    s = lax.axis_index("s")
    wid = s * NC + c
    zeros16 = jnp.zeros((16,), jnp.float32)

    def zb(r, carry):
        for k in range(D // 16):
            zbuf[r, pl.ds(k * 16, 16)] = zeros16
        return carry

    lax.fori_loop(0, ZROWS, zb, 0)

    def zs(i, carry):
        pltpu.sync_copy(zbuf, acc.at[pl.ds(s * ROWS_PER_TILE + i * ZROWS, ZROWS)])
        return carry

    lax.fori_loop(0, ROWS_PER_TILE // ZROWS, zs, 0)
    plsc.subcore_barrier()

    base = wid * EPW_PAD

    def chunk(i, carry):
        off = base + i * K
        pltpu.sync_copy(src_hbm.at[pl.ds(off, K)], sidx)
        pltpu.sync_copy(dst_hbm.at[pl.ds(off, K)], didx)
        pltpu.async_copy(g_hbm.at[sidx], rows, sem).wait()
        pltpu.sync_copy(rows, acc.at[didx], add=True)
        return carry

    lax.fori_loop(0, NCHUNK, chunk, 0)
    plsc.subcore_barrier()

    @pl.when(s == 0)
    def _():
        pltpu.sync_copy(acc, out_hbm.at[c])


def _mm1_body(x_ref, w_ref, degs_ref, g_ref, dinv_ref):
    deg = jnp.sum(degs_ref[...], axis=1, keepdims=True) + 1.0
    dinv = lax.rsqrt(deg)
    h = jnp.dot(x_ref[...], w_ref[...], preferred_element_type=jnp.float32)
    g_ref[...] = h * dinv
    dinv_ref[...] = dinv


def _mid_body(p_ref, g_ref, dinv_ref, b_ref, w_ref, o_ref):
    acc = p_ref[0] + p_ref[1] + g_ref[...]
    pre = acc * dinv_ref[...] + b_ref[...]
    hin = jnp.maximum(pre, 0.0)
    o_ref[...] = jnp.dot(hin, w_ref[...], preferred_element_type=jnp.float32) * dinv_ref[...]


def _fin_body(p_ref, g_ref, dinv_ref, b_ref, o_ref):
    acc = p_ref[0] + p_ref[1] + g_ref[...]
    o_ref[...] = acc * dinv_ref[...] + b_ref[...]


_row_spec = pl.BlockSpec((BLK, D), lambda i: (i, 0))
_dinv_spec = pl.BlockSpec((BLK, 1), lambda i: (i, 0))
_w_spec = pl.BlockSpec((D, D), lambda i: (0, 0))
_b_spec = pl.BlockSpec((1, D), lambda i: (0, 0))
_p_spec = pl.BlockSpec((NC, BLK, D), lambda i: (0, i, 0))

_mm1 = pl.pallas_call(
    _mm1_body,
    grid=(GRID,),
    in_specs=[_row_spec, _w_spec, pl.BlockSpec((BLK, NW), lambda i: (i, 0))],
    out_specs=[_row_spec, _dinv_spec],
    out_shape=[
        jax.ShapeDtypeStruct((N, D), jnp.float32),
        jax.ShapeDtypeStruct((N, 1), jnp.float32),
    ],
)

_mid = pl.pallas_call(
    _mid_body,
    grid=(GRID,),
    in_specs=[_p_spec, _row_spec, _dinv_spec, _b_spec, _w_spec],
    out_specs=_row_spec,
    out_shape=jax.ShapeDtypeStruct((N, D), jnp.float32),
)

_fin = pl.pallas_call(
    _fin_body,
    grid=(GRID,),
    in_specs=[_p_spec, _row_spec, _dinv_spec, _b_spec],
    out_specs=_row_spec,
    out_shape=jax.ShapeDtypeStruct((N, D), jnp.float32),
)


def kernel(x, edge_index, W1, b1, W2, b2, W3, b3):
    src = edge_index[0].astype(jnp.int32)
    dst = edge_index[1].astype(jnp.int32)
    pad = EPW_PAD - EPW
    src_p = jnp.concatenate(
        [src.reshape(NW, EPW), jnp.zeros((NW, pad), jnp.int32)], axis=1
    ).reshape(-1)
    dst_p = jnp.concatenate(
        [dst.reshape(NW, EPW), jnp.full((NW, pad), N, jnp.int32)], axis=1
    ).reshape(-1)
    b1r = b1.reshape(1, D)
    b2r = b2.reshape(1, D)
    b3r = b3.reshape(1, D)

    degs_t = _deg_kernel(dst).T
    g1, dinv = _mm1(x, W1, degs_t)
    p1 = _scatter_kernel(g1, src_p, dst_p)
    g2 = _mid(p1, g1, dinv, b1r, W2)
    p2 = _scatter_kernel(g2, src_p, dst_p)
    g3 = _mid(p2, g2, dinv, b2r, W3)
    p3 = _scatter_kernel(g3, src_p, dst_p)
    return _fin(p3, g3, dinv, b3r)


# speedup vs baseline: 8.3439x; 1.1322x over previous
import functools

import jax
import jax.numpy as jnp
from jax import lax
from jax.experimental import pallas as pl
from jax.experimental.pallas import tpu as pltpu
from jax.experimental.pallas import tpu_sc as plsc

N = 10000
D = 128
E = 320000
NC = 2
NS = 16
NW = NC * NS
EPW = E // NW
K = 128
EPW_PAD = 10240
NCHUNK = EPW_PAD // K
PAD_N = 10240
ROWS_PER_TILE = PAD_N // NS


BLK = 1000
GRID = N // BLK

_mesh = plsc.VectorSubcoreMesh(core_axis_name="c", subcore_axis_name="s")
_sc_params = pltpu.CompilerParams(needs_layout_passes=False)


@functools.partial(
    pl.kernel,
    mesh=_mesh,
    out_type=jax.ShapeDtypeStruct((NW, N), jnp.float32),
    scratch_types=[
        pltpu.VMEM((EPW,), jnp.int32),
        pltpu.VMEM((N,), jnp.float32),
    ],
    compiler_params=_sc_params,
)
def _deg_kernel(dst_hbm, out_hbm, idx_v, deg_v):
    c = lax.axis_index("c")
    s = lax.axis_index("s")
    wid = s * NC + c
    pltpu.sync_copy(dst_hbm.at[pl.ds(wid * EPW, EPW)], idx_v)
    zeros16 = jnp.zeros((16,), jnp.float32)
    ones16 = jnp.ones((16,), jnp.float32)

    def zbody(i, carry):
        deg_v[pl.ds(i * 16, 16)] = zeros16
        return carry

    lax.fori_loop(0, N // 16, zbody, 0)

    def body(i, carry):
        idx = idx_v[pl.ds(i * 16, 16)]
        plsc.addupdate_scatter(deg_v, [idx], ones16)
        return carry

    lax.fori_loop(0, EPW // 16, body, 0)
    pltpu.sync_copy(deg_v, out_hbm.at[wid])


@functools.partial(
    pl.kernel,
    mesh=_mesh,
    out_type=jax.ShapeDtypeStruct((NC, PAD_N, D), jnp.float32),
    scratch_types=[
        pltpu.VMEM((NCHUNK, K), jnp.int32),
        pltpu.VMEM((NCHUNK, K), jnp.int32),
        pltpu.VMEM((K, D), jnp.float32),
        pltpu.VMEM_SHARED((PAD_N, D), jnp.float32),
        pltpu.SemaphoreType.DMA,
    ],
    compiler_params=_sc_params,
)
def _scatter_kernel(g_hbm, src_hbm, dst_hbm, out_hbm,
                    sidx_all, didx_all, rows0, acc, sem):
    c_ax = lax.axis_index("c")
    s_ax = lax.axis_index("s")
    wid = s_ax * NC + c_ax
    zeros16 = jnp.zeros((16,), jnp.float32)

    def zb(r, carry):
        for k in range(D // 16):
            rows0[r, pl.ds(k * 16, 16)] = zeros16
        return carry

    lax.fori_loop(0, K, zb, 0)

    def zs(i, carry):
        pltpu.sync_copy(rows0, acc.at[pl.ds(s_ax * ROWS_PER_TILE + i * K, K)])
        return carry

    lax.fori_loop(0, ROWS_PER_TILE // K, zs, 0)

    pltpu.sync_copy(src_hbm.at[wid], sidx_all)
    pltpu.sync_copy(dst_hbm.at[wid], didx_all)
    plsc.subcore_barrier()

    def chunk(i, carry):
        pltpu.async_copy(g_hbm.at[sidx_all.at[i]], rows0, sem).wait()
        pltpu.sync_copy(rows0, acc.at[didx_all.at[i]], add=True)
        return carry

    lax.fori_loop(0, NCHUNK, chunk, 0)

    plsc.subcore_barrier()

    @pl.when(s_ax == 0)
    def _():
        pltpu.sync_copy(acc, out_hbm.at[c_ax])


def _mm1_body(x_ref, w_ref, degs_ref, g_ref, dinv_ref):
    deg = jnp.sum(degs_ref[...], axis=1, keepdims=True) + 1.0
    dinv = lax.rsqrt(deg)
    h = jnp.dot(x_ref[...], w_ref[...], preferred_element_type=jnp.float32)
    g_ref[...] = h * dinv
    dinv_ref[...] = dinv


def _mid_body(p_ref, g_ref, dinv_ref, b_ref, w_ref, o_ref):
    acc = p_ref[0] + p_ref[1] + g_ref[...]
    pre = acc * dinv_ref[...] + b_ref[...]
    hin = jnp.maximum(pre, 0.0)
    o_ref[...] = jnp.dot(hin, w_ref[...], preferred_element_type=jnp.float32) * dinv_ref[...]


def _fin_body(p_ref, g_ref, dinv_ref, b_ref, o_ref):
    acc = p_ref[0] + p_ref[1] + g_ref[...]
    o_ref[...] = acc * dinv_ref[...] + b_ref[...]


_row_spec = pl.BlockSpec((BLK, D), lambda i: (i, 0))
_dinv_spec = pl.BlockSpec((BLK, 1), lambda i: (i, 0))
_w_spec = pl.BlockSpec((D, D), lambda i: (0, 0))
_b_spec = pl.BlockSpec((1, D), lambda i: (0, 0))
_p_spec = pl.BlockSpec((NC, BLK, D), lambda i: (0, i, 0))

_mm1 = pl.pallas_call(
    _mm1_body,
    grid=(GRID,),
    in_specs=[_row_spec, _w_spec, pl.BlockSpec((BLK, NW), lambda i: (i, 0))],
    out_specs=[_row_spec, _dinv_spec],
    out_shape=[
        jax.ShapeDtypeStruct((N, D), jnp.float32),
        jax.ShapeDtypeStruct((N, 1), jnp.float32),
    ],
)

_mid = pl.pallas_call(
    _mid_body,
    grid=(GRID,),
    in_specs=[_p_spec, _row_spec, _dinv_spec, _b_spec, _w_spec],
    out_specs=_row_spec,
    out_shape=jax.ShapeDtypeStruct((N, D), jnp.float32),
)

_fin = pl.pallas_call(
    _fin_body,
    grid=(GRID,),
    in_specs=[_p_spec, _row_spec, _dinv_spec, _b_spec],
    out_specs=_row_spec,
    out_shape=jax.ShapeDtypeStruct((N, D), jnp.float32),
)


def kernel(x, edge_index, W1, b1, W2, b2, W3, b3):
    src = edge_index[0].astype(jnp.int32)
    dst = edge_index[1].astype(jnp.int32)
    pad = EPW_PAD - EPW
    src_p = jnp.concatenate(
        [src.reshape(NW, EPW), jnp.zeros((NW, pad), jnp.int32)], axis=1
    ).reshape(NW, NCHUNK, K)
    dst_p = jnp.concatenate(
        [dst.reshape(NW, EPW), jnp.full((NW, pad), N, jnp.int32)], axis=1
    ).reshape(NW, NCHUNK, K)
    b1r = b1.reshape(1, D)
    b2r = b2.reshape(1, D)
    b3r = b3.reshape(1, D)

    degs_t = _deg_kernel(dst).T
    g1, dinv = _mm1(x, W1, degs_t)
    p1 = _scatter_kernel(g1, src_p, dst_p)
    g2 = _mid(p1, g1, dinv, b1r, W2)
    p2 = _scatter_kernel(g2, src_p, dst_p)
    g3 = _mid(p2, g2, dinv, b2r, W3)
    p3 = _scatter_kernel(g3, src_p, dst_p)
    return _fin(p3, g3, dinv, b3r)
